# baseline (speedup 1.0000x reference)
import functools
import math

import jax
import jax.numpy as jnp
from jax import lax
from jax.experimental import pallas as pl
from jax.experimental.pallas import tpu as pltpu
from jax.experimental.pallas import tpu_sc as plsc

_NC = 2
_NS = 16
_NW = _NC * _NS
_TOK = 128
_LANES = 16
_GRING = 4
_ORING = 2
_AHEAD = 2
_ROW = 128
_STRIDE = 136


@jax.jit
def _run(xt, tt, tail128):
    S, B = xt.shape
    D, V = tt.shape
    scale = float(math.sqrt(D))
    mesh = plsc.VectorSubcoreMesh(core_axis_name="c", subcore_axis_name="s")

    n_full = V // _ROW // _NW * _NW
    n_tail_full = (V - n_full * _ROW) // _ROW
    v_part = n_full * _ROW + n_tail_full * _ROW
    n_part = V - v_part

    @functools.partial(
        pl.kernel,
        out_type=jax.ShapeDtypeStruct((V, _ROW), jnp.float32),
        mesh=mesh,
        compiler_params=pltpu.CompilerParams(needs_layout_passes=False),
        scratch_types=[
            pltpu.VMEM((2, D, _ROW), jnp.float32),
            pltpu.VMEM((2, _ROW, _STRIDE), jnp.float32),
            pltpu.SemaphoreType.DMA((2,)),
            pltpu.SemaphoreType.DMA((2,)),
        ],
    )
    def prep(tt_hbm, tail_hbm, out_hbm, ibuf, tbuf, sem_i, sem_o):
        wid = lax.axis_index("s") * _NC + lax.axis_index("c")

        def stage(v0, p):
            return pltpu.make_async_copy(
                tt_hbm.at[:, pl.ds(v0, _ROW)], ibuf.at[p], sem_i.at[p])

        def flush(v0, p):
            return pltpu.make_async_copy(
                tbuf.at[p, :, pl.ds(0, _ROW)],
                out_hbm.at[pl.ds(v0, _ROW)], sem_o.at[p])

        def v0_of(k):
            return (k * _NW + wid) * _ROW

        def transpose_block(p, n_vg):
            @pl.loop(0, D, unroll=4)
            def ff(f):
                fvec = jnp.zeros((_LANES,), jnp.int32) + f
                vs = [
                    ibuf[p, f, pl.ds(g * _LANES, _LANES)] * scale
                    for g in range(n_vg)
                ]
                for g in range(n_vg):
                    vidx = lax.iota(jnp.int32, _LANES) + jnp.int32(g * _LANES)
                    plsc.store_scatter(tbuf.at[p], [vidx, fvec], vs[g])

        n_blk = n_full // _NW
        stage(v0_of(0), 0).start()

        @pl.loop(0, n_blk)
        def blk(k):
            p = k % 2

            @pl.when(k + 1 < n_blk)
            def _():
                stage(v0_of(k + 1), (k + 1) % 2).start()

            stage(v0_of(k), p).wait()

            @pl.when(k >= 2)
            def _():
                flush(v0_of(k - 2), p).wait()

            transpose_block(p, _ROW // _LANES)
            flush(v0_of(k), p).start()

        for k in (n_blk - 2, n_blk - 1):
            flush(v0_of(k), k % 2).wait()

        @pl.when(wid < n_tail_full)
        def _():
            v0 = n_full * _ROW + wid * _ROW
            stage(v0, 0).start()
            stage(v0, 0).wait()
            transpose_block(0, _ROW // _LANES)
            flush(v0, 0).start()
            flush(v0, 0).wait()

        @pl.when(wid == n_tail_full)
        def _():
            pltpu.make_async_copy(tail_hbm, ibuf.at[0, :, :], sem_i.at[0]).start()
            pltpu.make_async_copy(tail_hbm, ibuf.at[0, :, :], sem_i.at[0]).wait()
            pltpu.make_async_copy(
                ibuf.at[0, :, :], out_hbm.at[pl.ds(v_part, n_part)],
                sem_o.at[0]).start()
            pltpu.make_async_copy(
                ibuf.at[0, :, :], out_hbm.at[pl.ds(v_part, n_part)],
                sem_o.at[0]).wait()

    trow = prep(tt, tail128)

    @functools.partial(
        pl.kernel,
        out_type=jax.ShapeDtypeStruct((S, D, B), jnp.float32),
        mesh=mesh,
        compiler_params=pltpu.CompilerParams(needs_layout_passes=False),
        scratch_types=[
            pltpu.VMEM((S, _TOK), jnp.int32),
            pltpu.VMEM((_GRING, _TOK, _ROW), jnp.float32),
            pltpu.VMEM((_ORING, D, _STRIDE), jnp.float32),
            pltpu.SemaphoreType.DMA((_GRING,)),
            pltpu.SemaphoreType.DMA((_ORING,)),
        ],
    )
    def look(xt_hbm, tab_hbm, out_hbm, idxs, gbuf, obuf, sem_g, sem_s):
        wid = lax.axis_index("s") * _NC + lax.axis_index("c")
        col = wid * _TOK
        pltpu.sync_copy(xt_hbm.at[:, pl.ds(col, _TOK)], idxs)

        def gather(r, q):
            return pltpu.make_async_copy(
                tab_hbm.at[idxs.at[r]], gbuf.at[q], sem_g.at[q])

        def scatter(r, qo):
            return pltpu.make_async_copy(
                obuf.at[qo, :, pl.ds(0, _TOK)],
                out_hbm.at[r, :, pl.ds(col, _TOK)], sem_s.at[qo])

        for rr in range(_AHEAD):
            gather(rr, rr % _GRING).start()

        @pl.loop(0, S)
        def chunk(r):
            q = r % _GRING
            qo = r % _ORING
            rg = r + _AHEAD

            @pl.when(rg < S)
            def _():
                gather(rg, rg % _GRING).start()

            gather(r, q).wait()

            @pl.when(r >= _ORING)
            def _():
                scatter(r - _ORING, qo).wait()

            @pl.loop(0, _TOK, unroll=4)
            def tok(t):
                tvec = jnp.zeros((_LANES,), jnp.int32) + t
                nk = D // _LANES
                vs = [gbuf[q, t, pl.ds(k * _LANES, _LANES)] for k in range(nk)]
                for k in range(nk):
                    fidx = lax.iota(jnp.int32, _LANES) + jnp.int32(k * _LANES)
                    plsc.store_scatter(obuf.at[qo], [fidx, tvec], vs[k])

            scatter(r, qo).start()

        for rr in range(_ORING):
            scatter(S - _ORING + rr, (S - _ORING + rr) % _ORING).wait()

    return look(xt, trow)


def kernel(x, table):
    xt = x.T.astype(jnp.int32)
    tt = table.T
    v_part = table.shape[0] // 128 * 128
    scale = math.sqrt(table.shape[1])
    tail128 = jnp.pad(table[v_part:] * scale,
                      ((0, 0), (0, 128 - table.shape[1])))
    out_t = _run(xt, tt, tail128)
    return jnp.transpose(out_t, (2, 0, 1))


# speedup vs baseline: 1.8193x; 1.8193x over previous
import functools
import math

import jax
import jax.numpy as jnp
from jax import lax
from jax.experimental import pallas as pl
from jax.experimental.pallas import tpu as pltpu
from jax.experimental.pallas import tpu_sc as plsc

_NC = 2
_NS = 16
_NW = _NC * _NS
_CHUNK = 128
_LANES = 16
_RING = 4
_AHEAD = 2


@jax.jit
def _lookup(x_flat, table):
    B = x_flat.shape[0]
    V, D = table.shape
    b_per_w = B // _NW
    n_chunks = b_per_w // _CHUNK
    scale = float(math.sqrt(D))
    mesh = plsc.VectorSubcoreMesh(core_axis_name="c", subcore_axis_name="s")

    idx2d = x_flat.reshape(_NW * n_chunks, _CHUNK)

    @functools.partial(
        pl.kernel,
        out_type=jax.ShapeDtypeStruct((B, D), jnp.float32),
        mesh=mesh,
        compiler_params=pltpu.CompilerParams(
            use_tc_tiling_on_sc=False, needs_layout_passes=False),
        scratch_types=[
            pltpu.VMEM((n_chunks, _CHUNK), jnp.int32),
            pltpu.VMEM((_RING, _CHUNK, D), jnp.float32),
            pltpu.SemaphoreType.DMA((_RING,)),
            pltpu.SemaphoreType.DMA((_RING,)),
        ],
    )
    def look(idx_hbm, table_hbm, out_hbm, idx_v, bufs, sem_g, sem_s):
        wid = lax.axis_index("s") * _NC + lax.axis_index("c")
        base = wid * b_per_w
        pltpu.sync_copy(idx_hbm.at[pl.ds(wid * n_chunks, n_chunks)], idx_v)

        def gather(j, b):
            return pltpu.make_async_copy(
                table_hbm.at[idx_v.at[j]], bufs.at[b], sem_g.at[b])

        def scatter(j, b):
            return pltpu.make_async_copy(
                bufs.at[b], out_hbm.at[pl.ds(base + j * _CHUNK, _CHUNK)],
                sem_s.at[b])

        for b in range(_AHEAD):
            gather(b, b).start()

        @pl.loop(0, n_chunks, step=_RING)
        def outer(j0):
            for b in range(_RING):
                j = j0 + b
                bb = (b + _AHEAD) % _RING
                jg = j + _AHEAD

                @pl.when(jg < n_chunks)
                def _():
                    @pl.when(jg >= _RING)
                    def _():
                        scatter(jg - _RING, bb).wait()

                    gather(jg, bb).start()

                gather(j, b).wait()

                @pl.loop(0, _CHUNK, unroll=8)
                def scale_body(i):
                    for t in range(D // _LANES):
                        sl = pl.ds(t * _LANES, _LANES)
                        bufs[b, i, sl] = bufs[b, i, sl] * scale

                scatter(j, b).start()

        for b in range(_RING):
            scatter(n_chunks - _RING + b, b).wait()

    return look(idx2d, table)


def kernel(x, table):
    out = _lookup(x.reshape(-1).astype(jnp.int32), table)
    return out.reshape(x.shape[0], x.shape[1], table.shape[1])
